# baseline (speedup 1.0000x reference)
import functools

import jax
import jax.numpy as jnp
from jax import lax
from jax.experimental import pallas as pl
from jax.experimental.pallas import tpu as pltpu
from jax.experimental.pallas import tpu_sc as plsc

N_TOK = 4096
K = 8192
C = 64
TOK_BLK = 512
K_CHUNK = 2048
N_BLKS = N_TOK // TOK_BLK
N_CHUNKS = K // K_CHUNK


def _argmin_body(x2_ref, xf_ref, cb_ref, idx_ref, loss_ref):
    xf = xf_ref[...]
    x2 = x2_ref[...]

    def chunk(j, carry):
        best_d, best_i = carry
        cb = cb_ref[pl.ds(j * K_CHUNK, K_CHUNK), :]
        c2 = jnp.sum(cb * cb, axis=1)
        dot = lax.dot_general(
            xf, cb, (((1,), (1,)), ((), ())),
            preferred_element_type=jnp.float32)
        dist2 = (x2 + c2[None, :]) - 2.0 * dot
        dist = jnp.sqrt(jnp.maximum(dist2, 0.0))
        dmin = jnp.min(dist, axis=1)
        amin = jnp.argmin(dist, axis=1).astype(jnp.int32) + j * K_CHUNK
        upd = dmin < best_d
        return jnp.where(upd, dmin, best_d), jnp.where(upd, amin, best_i)

    init = (jnp.full((TOK_BLK,), jnp.inf, jnp.float32),
            jnp.zeros((TOK_BLK,), jnp.int32))
    best_d, best_i = lax.fori_loop(0, N_CHUNKS, chunk, init)
    idx_ref[0, 0, :] = best_i
    loss_ref[...] = jnp.sum(best_d * best_d).reshape(1, 1, 1)


def _tc_argmin(x2, xf, codebook):
    return pl.pallas_call(
        _argmin_body,
        grid=(N_BLKS,),
        in_specs=[
            pl.BlockSpec((TOK_BLK, 1), lambda i: (i, 0)),
            pl.BlockSpec((TOK_BLK, C), lambda i: (i, 0)),
            pl.BlockSpec((K, C), lambda i: (0, 0)),
        ],
        out_specs=[
            pl.BlockSpec((1, 1, TOK_BLK), lambda i: (i, 0, 0)),
            pl.BlockSpec((1, 1, 1), lambda i: (i, 0, 0)),
        ],
        out_shape=[
            jax.ShapeDtypeStruct((N_BLKS, 1, TOK_BLK), jnp.int32),
            jax.ShapeDtypeStruct((N_BLKS, 1, 1), jnp.float32),
        ],
    )(x2, xf, codebook)


def _sc_gather(codebook, idx):
    info = plsc.get_sparse_core_info()
    nw = info.num_cores * info.num_subcores
    b_per_w = N_TOK // nw

    mesh = plsc.VectorSubcoreMesh(core_axis_name="c", subcore_axis_name="s")

    @functools.partial(
        pl.kernel,
        mesh=mesh,
        out_type=jax.ShapeDtypeStruct((N_TOK, C), jnp.float32),
        scratch_types=[
            pltpu.VMEM((b_per_w,), jnp.int32),
            pltpu.VMEM((b_per_w, C), jnp.float32),
            pltpu.SemaphoreType.DMA,
        ],
        compiler_params=pltpu.CompilerParams(use_tc_tiling_on_sc=False),
    )
    def gather(cb_hbm, idx_hbm, out_hbm, idx_v, rows_v, sem):
        wid = lax.axis_index("s") * info.num_cores + lax.axis_index("c")
        base = wid * b_per_w
        pltpu.sync_copy(idx_hbm.at[pl.ds(base, b_per_w)], idx_v)
        pltpu.async_copy(cb_hbm.at[idx_v], rows_v, sem).wait()
        pltpu.sync_copy(rows_v, out_hbm.at[pl.ds(base, b_per_w)])

    return gather(codebook, idx)


def kernel(x, codebook):
    B, Cc, H, W = x.shape
    xf3 = jnp.transpose(x, (0, 2, 3, 1)).reshape(B, H * W, Cc)
    x2 = jnp.sum(xf3 ** 2, axis=-1, keepdims=True)
    xf = xf3.reshape(-1, Cc)
    idx3, loss_parts = _tc_argmin(x2.reshape(-1, 1), xf, codebook)
    idx = idx3.reshape(-1)
    quant = jnp.take(codebook, idx, axis=0)
    loss = jnp.sum(loss_parts) / jnp.float32(N_TOK * Cc)
    quant_out = jnp.transpose(quant.reshape(B, H, W, Cc), (0, 3, 1, 2))
    min_encoding_indices = idx.reshape(B, H, W)
    return quant_out, loss, loss, min_encoding_indices


# speedup vs baseline: 1.2098x; 1.2098x over previous
import functools

import jax
import jax.numpy as jnp
from jax import lax
from jax.experimental import pallas as pl
from jax.experimental.pallas import tpu as pltpu
from jax.experimental.pallas import tpu_sc as plsc

N_TOK = 4096
K = 8192
C = 64
TOK_BLK = 512
K_CHUNK = 2048
N_BLKS = N_TOK // TOK_BLK
N_CHUNKS = K // K_CHUNK


def _f32_succ(x):
    return lax.bitcast_convert_type(
        lax.bitcast_convert_type(x, jnp.int32) + 1, jnp.float32)


def _f32_pred(x):
    return lax.bitcast_convert_type(
        lax.bitcast_convert_type(x, jnp.int32) - 1, jnp.float32)


def _g(x):
    return jnp.sqrt(jnp.maximum(x, 0.0))


def _argmin_body(x2_ref, xm_ref, cb_ref, c2_ref, idx_ref, loss_ref):
    xm = xm_ref[...]
    x2 = x2_ref[...]
    x2e = jnp.concatenate([x2, jnp.ones_like(x2)], axis=1)
    iota = lax.broadcasted_iota(jnp.int32, (TOK_BLK, K_CHUNK), 1)

    best_dq = jnp.full((TOK_BLK,), jnp.inf, jnp.float32)
    best_i = jnp.zeros((TOK_BLK,), jnp.int32)
    best_s = jnp.zeros((TOK_BLK,), jnp.float32)

    for j in range(N_CHUNKS):
        cb = cb_ref[pl.ds(j * K_CHUNK, K_CHUNK), :]
        c2 = c2_ref[pl.ds(j * K_CHUNK, K_CHUNK)]
        cr = jnp.concatenate(
            [jnp.ones((1, K_CHUNK), jnp.float32), c2[None, :]], axis=0)
        dotm = lax.dot_general(
            xm, cb, (((1,), (1,)), ((), ())),
            preferred_element_type=jnp.float32)
        bc = lax.dot_general(
            x2e, cr, (((1,), (0,)), ((), ())),
            preferred_element_type=jnp.float32)
        s = bc + dotm
        smin = jnp.min(s, axis=1)

        dq = _g(smin)
        dqs = _f32_succ(dq)
        t = dqs * dqs
        for _ in range(4):
            t = jnp.where(_g(t) > dq, _f32_pred(t), t)
        li = jnp.min(jnp.where(s <= t[:, None], iota, K), axis=1) + j * K_CHUNK

        upd = dq < best_dq
        best_dq = jnp.where(upd, dq, best_dq)
        best_i = jnp.where(upd, li, best_i)
        best_s = jnp.where(upd, smin, best_s)

    idx_ref[0, 0, :] = best_i
    loss_ref[...] = jnp.sum(jnp.maximum(best_s, 0.0)).reshape(1, 1, 1)


def _tc_argmin(x2, xm, codebook, c2):
    return pl.pallas_call(
        _argmin_body,
        grid=(N_BLKS,),
        in_specs=[
            pl.BlockSpec((TOK_BLK, 1), lambda i: (i, 0)),
            pl.BlockSpec((TOK_BLK, C), lambda i: (i, 0)),
            pl.BlockSpec((K, C), lambda i: (0, 0)),
            pl.BlockSpec((K,), lambda i: (0,)),
        ],
        out_specs=[
            pl.BlockSpec((1, 1, TOK_BLK), lambda i: (i, 0, 0)),
            pl.BlockSpec((1, 1, 1), lambda i: (i, 0, 0)),
        ],
        out_shape=[
            jax.ShapeDtypeStruct((N_BLKS, 1, TOK_BLK), jnp.int32),
            jax.ShapeDtypeStruct((N_BLKS, 1, 1), jnp.float32),
        ],
    )(x2, xm, codebook, c2)


def _sc_gather(codebook, idx):
    info = plsc.get_sparse_core_info()
    nw = info.num_cores * info.num_subcores
    b_per_w = N_TOK // nw

    mesh = plsc.VectorSubcoreMesh(core_axis_name="c", subcore_axis_name="s")

    @functools.partial(
        pl.kernel,
        mesh=mesh,
        out_type=jax.ShapeDtypeStruct((N_TOK, C), jnp.float32),
        scratch_types=[
            pltpu.VMEM((b_per_w,), jnp.int32),
            pltpu.VMEM((b_per_w, C), jnp.float32),
            pltpu.SemaphoreType.DMA,
        ],
        compiler_params=pltpu.CompilerParams(use_tc_tiling_on_sc=False),
    )
    def gather(cb_hbm, idx_hbm, out_hbm, idx_v, rows_v, sem):
        wid = lax.axis_index("s") * info.num_cores + lax.axis_index("c")
        base = wid * b_per_w
        pltpu.sync_copy(idx_hbm.at[pl.ds(base, b_per_w)], idx_v)
        pltpu.async_copy(cb_hbm.at[idx_v], rows_v, sem).wait()
        pltpu.sync_copy(rows_v, out_hbm.at[pl.ds(base, b_per_w)])

    return gather(codebook, idx)


def kernel(x, codebook):
    B, Cc, H, W = x.shape
    xf3 = jnp.transpose(x, (0, 2, 3, 1)).reshape(B, H * W, Cc)
    x2 = jnp.sum(xf3 ** 2, axis=-1, keepdims=True)
    c2 = jnp.sum(codebook ** 2, axis=-1)
    xm = (-2.0 * xf3).reshape(-1, Cc)
    idx3, loss_parts = _tc_argmin(x2.reshape(-1, 1), xm, codebook, c2)
    idx = idx3.reshape(-1)
    quant = _sc_gather(codebook, idx)
    loss = jnp.sum(loss_parts) / jnp.float32(N_TOK * Cc)
    quant_out = jnp.transpose(quant.reshape(B, H, W, Cc), (0, 3, 1, 2))
    min_encoding_indices = idx.reshape(B, H, W)
    return quant_out, loss, loss, min_encoding_indices
